# baseline (speedup 1.0000x reference)
import functools

import jax
import jax.numpy as jnp
from jax import lax
from jax.experimental import pallas as pl
from jax.experimental.pallas import tpu as pltpu
from jax.experimental.pallas import tpu_sc as plsc

_CHUNK = 128
_VB = 2048


def _h_body(x_ref, w1_ref, b1_ref, h_ref):
    acc = lax.dot_general(
        x_ref[...], w1_ref[...], (((1,), (1,)), ((), ())),
        preferred_element_type=jnp.float32)
    h_ref[...] = jnp.maximum(acc + b1_ref[...], 0.0)


def _stats_body(v_total, h_ref, w2_ref, b2_ref, norm_ref, m_scr, s_scr):
    j = pl.program_id(0)
    logits = lax.dot_general(
        h_ref[...], w2_ref[...], (((1,), (1,)), ((), ())),
        preferred_element_type=jnp.float32) + b2_ref[0]
    col = j * _VB + lax.broadcasted_iota(jnp.int32, (1, _VB), 1)
    logits = jnp.where(col < v_total, logits, -jnp.inf)
    tmax = jnp.max(logits, axis=1, keepdims=True)

    @pl.when(j == 0)
    def _():
        m_scr[...] = tmax
        s_scr[...] = jnp.sum(jnp.exp(logits - tmax), axis=1, keepdims=True)

    @pl.when(j > 0)
    def _():
        m_old = m_scr[...]
        m_new = jnp.maximum(m_old, tmax)
        s_scr[...] = (s_scr[...] * jnp.exp(m_old - m_new)
                      + jnp.sum(jnp.exp(logits - m_new), axis=1,
                                keepdims=True))
        m_scr[...] = m_new

    @pl.when(j == pl.num_programs(0) - 1)
    def _():
        norm_ref[...] = m_scr[...] + jnp.log(s_scr[...])


def _out_body(h_ref, w2_ref, b2_ref, norm_ref, out_ref):
    logits = lax.dot_general(
        h_ref[...], w2_ref[...], (((1,), (1,)), ((), ())),
        preferred_element_type=jnp.float32) + b2_ref[0]
    out_ref[...] = logits - norm_ref[...]


@functools.cache
def _make_gather(nw, nc, n_chunk, emb_dim):
    mesh = plsc.VectorSubcoreMesh(core_axis_name="c", subcore_axis_name="s")

    @functools.partial(
        pl.kernel,
        out_type=jax.ShapeDtypeStruct((nw, n_chunk, _CHUNK, emb_dim),
                                      jnp.float32),
        mesh=mesh,
        scratch_types=[
            pltpu.VMEM((n_chunk, _CHUNK), jnp.int32),
            pltpu.VMEM((n_chunk, _CHUNK, emb_dim), jnp.float32),
            pltpu.SemaphoreType.DMA,
        ],
    )
    def gather_k(idx_hbm, table_hbm, out_hbm, idx_v, rows_v, sem):
        wid = lax.axis_index("s") * nc + lax.axis_index("c")
        pltpu.sync_copy(idx_hbm.at[wid], idx_v)
        copies = [
            pltpu.async_copy(table_hbm.at[idx_v.at[j]], rows_v.at[j], sem)
            for j in range(n_chunk)
        ]
        for c in copies:
            c.wait()
        pltpu.sync_copy(rows_v, out_hbm.at[wid])

    return gather_k


def kernel(inputs, emb, W1, b1, W2, b2):
    B, CTX = inputs.shape
    V, E = emb.shape
    H = W1.shape[0]
    R = B * CTX

    info = plsc.get_sparse_core_info()
    nw = info.num_cores * info.num_subcores
    n_chunk = R // (nw * _CHUNK)

    idx = inputs.astype(jnp.int32).reshape(nw, n_chunk, _CHUNK)
    emb_p = jnp.pad(emb, ((0, 0), (0, 128 - E)))
    gathered = _make_gather(nw, info.num_cores, n_chunk, 128)(idx, emb_p)
    x = gathered.reshape(R, 128)[:, :E].reshape(B, CTX * E)

    h = pl.pallas_call(
        _h_body,
        out_shape=jax.ShapeDtypeStruct((B, H), jnp.float32),
    )(x, W1, b1.reshape(1, H))

    nv = (V + _VB - 1) // _VB
    b2r = jnp.pad(b2, (0, nv * _VB - V)).reshape(nv, 1, _VB)

    norm = pl.pallas_call(
        functools.partial(_stats_body, V),
        grid=(nv,),
        in_specs=[
            pl.BlockSpec((B, H), lambda j: (0, 0)),
            pl.BlockSpec((_VB, H), lambda j: (j, 0)),
            pl.BlockSpec((1, 1, _VB), lambda j: (j, 0, 0)),
        ],
        out_specs=pl.BlockSpec((B, 1), lambda j: (0, 0)),
        out_shape=jax.ShapeDtypeStruct((B, 1), jnp.float32),
        scratch_shapes=[
            pltpu.VMEM((B, 1), jnp.float32),
            pltpu.VMEM((B, 1), jnp.float32),
        ],
        compiler_params=pltpu.CompilerParams(
            dimension_semantics=("arbitrary",)),
    )(h, W2, b2r)

    out = pl.pallas_call(
        _out_body,
        grid=(nv,),
        in_specs=[
            pl.BlockSpec((B, H), lambda j: (0, 0)),
            pl.BlockSpec((_VB, H), lambda j: (j, 0)),
            pl.BlockSpec((1, 1, _VB), lambda j: (j, 0, 0)),
            pl.BlockSpec((B, 1), lambda j: (0, 0)),
        ],
        out_specs=pl.BlockSpec((B, _VB), lambda j: (0, j)),
        out_shape=jax.ShapeDtypeStruct((B, V), jnp.float32),
        compiler_params=pltpu.CompilerParams(
            dimension_semantics=("arbitrary",)),
    )(h, W2, b2r, norm)

    return out
